# baseline (speedup 1.0000x reference)
import functools

import jax
import jax.numpy as jnp
from jax import lax
from jax.experimental import pallas as pl
from jax.experimental.pallas import tpu as pltpu
from jax.experimental.pallas import tpu_sc as plsc

N = 10000
E = 320000
D = 128

NC = 2
NS = 16
NW = NC * NS
C = 128
NPASS = 5
PCHUNK = 16
EPT = NPASS * PCHUNK * C
E_PAD = NW * EPT
N_PAD = 10240
ROWS_PT = N_PAD // NS


def _mm_body(x_ref, w_ref, o_ref):
    o_ref[...] = jnp.dot(x_ref[...], w_ref[...],
                         preferred_element_type=jnp.float32)


def _matmul(x, w):
    MB = 1000
    return pl.pallas_call(
        _mm_body,
        grid=(N // MB,),
        in_specs=[pl.BlockSpec((MB, D), lambda i: (i, 0)),
                  pl.BlockSpec((D, D), lambda i: (0, 0))],
        out_specs=pl.BlockSpec((MB, D), lambda i: (i, 0)),
        out_shape=jax.ShapeDtypeStruct((N, D), jnp.float32),
    )(x, w)


def _comb_body(p_ref, b_ref, o_ref):
    o_ref[...] = p_ref[0] + p_ref[1] + b_ref[0:1]


def _combine(partials, bias):
    MB = 1000
    bias8 = jnp.broadcast_to(bias.reshape(1, D), (8, D))
    return pl.pallas_call(
        _comb_body,
        grid=(N // MB,),
        in_specs=[pl.BlockSpec((2, MB, D), lambda i: (0, i, 0)),
                  pl.BlockSpec((8, D), lambda i: (0, 0))],
        out_specs=pl.BlockSpec((MB, D), lambda i: (i, 0)),
        out_shape=jax.ShapeDtypeStruct((N, D), jnp.float32),
    )(partials, bias8)


_mesh = plsc.VectorSubcoreMesh(core_axis_name="c", subcore_axis_name="s")


@functools.partial(
    pl.kernel,
    out_type=jax.ShapeDtypeStruct((NC, N_PAD, D), jnp.float32),
    mesh=_mesh,
    scratch_types=[
        pltpu.VMEM_SHARED((N_PAD, D), jnp.float32),
        pltpu.VMEM((PCHUNK, 2, C), jnp.int32),
        pltpu.VMEM((PCHUNK, C), jnp.float32),
        pltpu.VMEM((C, D), jnp.float32),
        pltpu.VMEM((C, D), jnp.float32),
        pltpu.SemaphoreType.DMA,
        pltpu.SemaphoreType.DMA,
    ],
)
def _sc_edges(sup, rc, ew, out, acc, rc_v, w_v, rows0, rows1, sem0, sem1):
    c = lax.axis_index("c")
    s = lax.axis_index("s")
    wid = s * NC + c

    zero16 = jnp.zeros((16,), jnp.float32)

    def _zrow(r, _):
        for j in range(D // 16):
            rows0[r, pl.ds(j * 16, 16)] = zero16
        return 0

    lax.fori_loop(0, C, _zrow, 0)
    for k in range(ROWS_PT // C):
        pltpu.sync_copy(rows0, acc.at[pl.ds(s * ROWS_PT + k * C, C)])
    plsc.subcore_barrier()

    def _scale(buf, k):
        def _g(g, _):
            wvec = w_v[k, pl.ds(g * 16, 16)]
            for t in range(16):
                e = g * 16 + t
                w_e = wvec[t]
                for j in range(D // 16):
                    sl = pl.ds(j * 16, 16)
                    buf[e, sl] = buf[e, sl] * w_e
            return 0

        lax.fori_loop(0, C // 16, _g, 0)

    def _pair(i, _):
        k0 = 2 * i
        k1 = 2 * i + 1
        pltpu.make_async_copy(sup.at[rc_v.at[k0, 0]], rows0, sem0).wait()
        pltpu.async_copy(sup.at[rc_v.at[k1, 0]], rows1, sem1)
        _scale(rows0, k0)
        pltpu.sync_copy(rows0, acc.at[rc_v.at[k0, 1]], add=True)

        @pl.when(k0 + 2 < PCHUNK)
        def _():
            pltpu.async_copy(sup.at[rc_v.at[k0 + 2, 0]], rows0, sem0)

        pltpu.make_async_copy(sup.at[rc_v.at[k1, 0]], rows1, sem1).wait()
        _scale(rows1, k1)
        pltpu.sync_copy(rows1, acc.at[rc_v.at[k1, 1]], add=True)
        return 0

    for p in range(NPASS):
        pltpu.sync_copy(rc.at[wid, p], rc_v)
        pltpu.sync_copy(ew.at[wid, p], w_v)
        pltpu.async_copy(sup.at[rc_v.at[0, 0]], rows0, sem0)
        lax.fori_loop(0, PCHUNK // 2, _pair, 0)

    plsc.subcore_barrier()

    pltpu.sync_copy(acc.at[pl.ds(s * ROWS_PT, ROWS_PT)],
                    out.at[c, pl.ds(s * ROWS_PT, ROWS_PT)])


def kernel(x, edge_index, edge_weight, weight, bias):
    npad = E_PAD - E
    row = jnp.concatenate(
        [edge_index[0].astype(jnp.int32), jnp.zeros((npad,), jnp.int32)])
    col = jnp.concatenate(
        [edge_index[1].astype(jnp.int32), jnp.zeros((npad,), jnp.int32)])
    ew = jnp.concatenate([edge_weight, jnp.zeros((npad,), jnp.float32)])
    row = row.reshape(NW, NPASS, PCHUNK, C)
    col = col.reshape(NW, NPASS, PCHUNK, C)
    rc = jnp.stack([row, col], axis=3)
    ew = ew.reshape(NW, NPASS, PCHUNK, C)
    support = _matmul(x, weight)
    partials = _sc_edges(support, rc, ew)
    return _combine(partials, bias)
